# baseline (speedup 1.0000x reference)
import functools

import jax
import jax.numpy as jnp
from jax import lax
from jax.experimental import pallas as pl
from jax.experimental.pallas import tpu as pltpu
from jax.experimental.pallas import tpu_sc as plsc

_V = 26
_C = 16
_S = 100000
_B = 4096
_ROWS = _V * _C
_NC = 2
_NS = 16
_NW = _NC * _NS
_RPW = _ROWS // _NW
_LANES = 16
_TINY = 1.1754943508222875e-38
_LN2 = 0.6931471805599453

_P5 = 0.04392862784795105
_P4 = -0.409475585766429
_P3 = 1.610177546896671
_P2 = -3.520218838145311
_P1 = 5.0697563166331205
_P0 = -2.7941536765360095


def _sc_body(
    xt_hbm, w2d_hbm, w_hbm, si_hbm, xcol, rowbuf, wbuf, sibuf, sem
):
    wid = lax.axis_index("s") * _NC + lax.axis_index("c")

    def row_body(r, vprev):
        row = wid * _RPW + r
        v = row // _C

        @pl.when(v != vprev)
        def _():
            pltpu.sync_copy(xt_hbm.at[v], xcol)

        pltpu.sync_copy(w2d_hbm.at[row], rowbuf)

        p_ = r % 2

        @pl.when(r >= 2)
        def _():
            pltpu.make_async_copy(wbuf.at[p_], w_hbm.at[row - 2], sem).wait()
            pltpu.make_async_copy(sibuf.at[p_], si_hbm.at[row - 2], sem).wait()

        def body(j, carry):
            sl = pl.ds(j * _LANES, _LANES)
            s = xcol[sl]
            g = plsc.load_gather(rowbuf, [s])
            si = jnp.sign(g)
            a = jnp.maximum(jnp.abs(g), _TINY)
            bi = plsc.bitcast(a, jnp.int32)
            e = (bi >> 23) - 127
            m = plsc.bitcast(
                (bi & 0x007FFFFF) | 0x3F800000, jnp.float32
            )
            p = _P5 * m + _P4
            p = p * m + _P3
            p = p * m + _P2
            p = p * m + _P1
            p = p * m + _P0
            w = (e.astype(jnp.float32) + p) * _LN2
            wbuf[p_, sl] = w
            sibuf[p_, sl] = si
            return carry

        lax.fori_loop(0, _B // _LANES, body, 0)

        pltpu.async_copy(wbuf.at[p_], w_hbm.at[row], sem)
        pltpu.async_copy(sibuf.at[p_], si_hbm.at[row], sem)
        return v

    lax.fori_loop(0, _RPW, row_body, -1)

    last = wid * _RPW + _RPW - 1
    pltpu.make_async_copy(wbuf.at[(_RPW - 2) % 2], w_hbm.at[last - 1], sem).wait()
    pltpu.make_async_copy(sibuf.at[(_RPW - 2) % 2], si_hbm.at[last - 1], sem).wait()
    pltpu.make_async_copy(wbuf.at[(_RPW - 1) % 2], w_hbm.at[last], sem).wait()
    pltpu.make_async_copy(sibuf.at[(_RPW - 1) % 2], si_hbm.at[last], sem).wait()


_sc_lookup = functools.partial(
    pl.kernel,
    out_type=(
        jax.ShapeDtypeStruct((_ROWS, _B), jnp.float32),
        jax.ShapeDtypeStruct((_ROWS, _B), jnp.float32),
    ),
    mesh=plsc.VectorSubcoreMesh(core_axis_name="c", subcore_axis_name="s"),
    scratch_types=[
        pltpu.VMEM((_B,), jnp.int32),
        pltpu.VMEM((_S,), jnp.float32),
        pltpu.VMEM((2, _B), jnp.float32),
        pltpu.VMEM((2, _B), jnp.float32),
        pltpu.SemaphoreType.DMA,
    ],
    compiler_params=pltpu.CompilerParams(needs_layout_passes=False),
)(_sc_body)


def kernel(x, weight):
    xt = x.T
    w2d = weight.reshape(_ROWS, _S)
    w, si = _sc_lookup(xt, w2d)
    w = w.T.reshape(_B, _V, 1, _C)
    si = si.T.reshape(_B, _V, 1, _C)
    return (w, si)


# speedup vs baseline: 1.8074x; 1.8074x over previous
import functools

import jax
import jax.numpy as jnp
from jax import lax
from jax.experimental import pallas as pl
from jax.experimental.pallas import tpu as pltpu
from jax.experimental.pallas import tpu_sc as plsc

_V = 26
_C = 16
_S = 100000
_B = 4096
_ROWS = _V * _C
_NC = 2
_NS = 16
_NW = _NC * _NS
_RPW = _ROWS // _NW
_LANES = 16
_TINY = 1.1754943508222875e-38
_LN2 = 0.6931471805599453

_P5 = 0.04392862784795105
_P4 = -0.409475585766429
_P3 = 1.610177546896671
_P2 = -3.520218838145311
_P1 = 5.0697563166331205
_P0 = -2.7941536765360095


_H0 = 50048
_H1 = _S - _H0


def _sc_body(
    xt_hbm, w2d_hbm, w_hbm, si_hbm, xcol, rowa, rowb, gbuf, wbuf, sibuf,
    rsem, osem
):
    wid = lax.axis_index("s") * _NC + lax.axis_index("c")
    row0 = wid * _RPW

    def fire0(row):
        pltpu.async_copy(w2d_hbm.at[row].at[pl.ds(0, _H0)], rowa, rsem)

    def wait0(row):
        pltpu.make_async_copy(
            w2d_hbm.at[row].at[pl.ds(0, _H0)], rowa, rsem
        ).wait()

    def fire1(row):
        pltpu.async_copy(w2d_hbm.at[row].at[pl.ds(_H0, _H1)], rowb, rsem)

    def wait1(row):
        pltpu.make_async_copy(
            w2d_hbm.at[row].at[pl.ds(_H0, _H1)], rowb, rsem
        ).wait()

    pltpu.sync_copy(xt_hbm.at[row0 // _C], xcol)
    fire0(row0)

    def row_body(r, vprev):
        row = row0 + r
        p_ = r % 2

        wait0(row)
        fire1(row)

        @plsc.parallel_loop(0, _B // _LANES, unroll=8)
        def body0(j):
            sl = pl.ds(j * _LANES, _LANES)
            s = xcol[sl]
            sc_ = jnp.minimum(s, _H0 - 1)
            gbuf[sl] = plsc.load_gather(rowa, [sc_])

        wait1(row)

        @pl.when(r < _RPW - 1)
        def _():
            fire0(row + 1)

        @pl.when(r >= 2)
        def _():
            pltpu.make_async_copy(wbuf.at[p_], w_hbm.at[row - 2], osem).wait()
            pltpu.make_async_copy(sibuf.at[p_], si_hbm.at[row - 2], osem).wait()

        @plsc.parallel_loop(0, _B // _LANES, unroll=4)
        def body1(j):
            sl = pl.ds(j * _LANES, _LANES)
            s = xcol[sl]
            hi = s >= _H0
            sc_ = jnp.clip(s - _H0, 0, _H1 - 1)
            g1 = plsc.load_gather(rowb, [sc_])
            g = jnp.where(hi, g1, gbuf[sl])
            one = jnp.float32(1.0)
            si = jnp.where(g > 0, one, jnp.float32(0.0)) - jnp.where(
                g < 0, one, jnp.float32(0.0)
            )
            a = jnp.maximum(jnp.abs(g), _TINY)
            bi = plsc.bitcast(a, jnp.int32)
            e = (bi >> 23) - 127
            m = plsc.bitcast(
                (bi & 0x007FFFFF) | 0x3F800000, jnp.float32
            )
            p = _P5 * m + _P4
            p = p * m + _P3
            p = p * m + _P2
            p = p * m + _P1
            p = p * m + _P0
            w = (e.astype(jnp.float32) + p) * _LN2
            wbuf[p_, sl] = w
            sibuf[p_, sl] = si

        pltpu.async_copy(wbuf.at[p_], w_hbm.at[row], osem)
        pltpu.async_copy(sibuf.at[p_], si_hbm.at[row], osem)

        nv = (row + 1) // _C
        @pl.when(nv != vprev)
        def _():
            pltpu.sync_copy(xt_hbm.at[jnp.minimum(nv, _V - 1)], xcol)

        return nv

    lax.fori_loop(0, _RPW, row_body, row0 // _C)

    last = row0 + _RPW - 1
    pltpu.make_async_copy(wbuf.at[(_RPW - 2) % 2], w_hbm.at[last - 1], osem).wait()
    pltpu.make_async_copy(sibuf.at[(_RPW - 2) % 2], si_hbm.at[last - 1], osem).wait()
    pltpu.make_async_copy(wbuf.at[(_RPW - 1) % 2], w_hbm.at[last], osem).wait()
    pltpu.make_async_copy(sibuf.at[(_RPW - 1) % 2], si_hbm.at[last], osem).wait()


_sc_lookup = functools.partial(
    pl.kernel,
    out_type=(
        jax.ShapeDtypeStruct((_ROWS, _B), jnp.float32),
        jax.ShapeDtypeStruct((_ROWS, _B), jnp.float32),
    ),
    mesh=plsc.VectorSubcoreMesh(core_axis_name="c", subcore_axis_name="s"),
    scratch_types=[
        pltpu.VMEM((_B,), jnp.int32),
        pltpu.VMEM((_H0,), jnp.float32),
        pltpu.VMEM((_H1,), jnp.float32),
        pltpu.VMEM((_B,), jnp.float32),
        pltpu.VMEM((2, _B), jnp.float32),
        pltpu.VMEM((2, _B), jnp.float32),
        pltpu.SemaphoreType.DMA,
        pltpu.SemaphoreType.DMA,
    ],
    compiler_params=pltpu.CompilerParams(needs_layout_passes=False),
)(_sc_body)


def kernel(x, weight):
    xt = x.T
    w2d = weight.reshape(_ROWS, _S)
    w, si = _sc_lookup(xt, w2d)
    w = w.T.reshape(_B, _V, 1, _C)
    si = si.T.reshape(_B, _V, 1, _C)
    return (w, si)


# speedup vs baseline: 1.8114x; 1.0022x over previous
import functools

import jax
import jax.numpy as jnp
from jax import lax
from jax.experimental import pallas as pl
from jax.experimental.pallas import tpu as pltpu
from jax.experimental.pallas import tpu_sc as plsc

_V = 26
_C = 16
_S = 100000
_B = 4096
_ROWS = _V * _C
_NC = 2
_NS = 16
_NW = _NC * _NS
_RPW = _ROWS // _NW
_LANES = 16
_TINY = 1.1754943508222875e-38
_LN2 = 0.6931471805599453

_P5 = 0.04392862784795105
_P4 = -0.409475585766429
_P3 = 1.610177546896671
_P2 = -3.520218838145311
_P1 = 5.0697563166331205
_P0 = -2.7941536765360095


_H0 = 50048
_H1 = _S - _H0


def _sc_body(
    xt_hbm, w2d_hbm, w_hbm, si_hbm, xcol, rowa, rowb, gbuf, wbuf, sibuf,
    rsem, osem
):
    wid = lax.axis_index("s") * _NC + lax.axis_index("c")
    row0 = wid * _RPW

    def fire0(row):
        pltpu.async_copy(w2d_hbm.at[row].at[pl.ds(0, _H0)], rowa, rsem)

    def wait0(row):
        pltpu.make_async_copy(
            w2d_hbm.at[row].at[pl.ds(0, _H0)], rowa, rsem
        ).wait()

    def fire1(row):
        pltpu.async_copy(w2d_hbm.at[row].at[pl.ds(_H0, _H1)], rowb, rsem)

    def wait1(row):
        pltpu.make_async_copy(
            w2d_hbm.at[row].at[pl.ds(_H0, _H1)], rowb, rsem
        ).wait()

    pltpu.sync_copy(xt_hbm.at[row0 // _C], xcol)
    fire0(row0)

    def row_body(r, vprev):
        row = row0 + r
        p_ = r % 2

        wait0(row)
        fire1(row)

        @plsc.parallel_loop(0, _B // _LANES, unroll=16)
        def body0(j):
            sl = pl.ds(j * _LANES, _LANES)
            s = xcol[sl]
            sc_ = jnp.minimum(s, _H0 - 1)
            gbuf[sl] = plsc.load_gather(rowa, [sc_])

        wait1(row)

        @pl.when(r < _RPW - 1)
        def _():
            fire0(row + 1)

        @pl.when(r >= 2)
        def _():
            pltpu.make_async_copy(wbuf.at[p_], w_hbm.at[row - 2], osem).wait()
            pltpu.make_async_copy(sibuf.at[p_], si_hbm.at[row - 2], osem).wait()

        @plsc.parallel_loop(0, _B // _LANES, unroll=8)
        def body1(j):
            sl = pl.ds(j * _LANES, _LANES)
            s = xcol[sl]
            hi = s >= _H0
            sc_ = jnp.clip(s - _H0, 0, _H1 - 1)
            g1 = plsc.load_gather(rowb, [sc_])
            g = jnp.where(hi, g1, gbuf[sl])
            gb = plsc.bitcast(g, jnp.int32)
            sgn = plsc.bitcast(
                (gb & jnp.int32(-2147483648)) | 0x3F800000, jnp.float32
            )
            si = jnp.where(g == 0, jnp.float32(0.0), sgn)
            a = jnp.maximum(jnp.abs(g), _TINY)
            bi = plsc.bitcast(a, jnp.int32)
            e = (bi >> 23) - 127
            m = plsc.bitcast(
                (bi & 0x007FFFFF) | 0x3F800000, jnp.float32
            )
            p = _P5 * m + _P4
            p = p * m + _P3
            p = p * m + _P2
            p = p * m + _P1
            p = p * m + _P0
            w = (e.astype(jnp.float32) + p) * _LN2
            wbuf[p_, sl] = w
            sibuf[p_, sl] = si

        pltpu.async_copy(wbuf.at[p_], w_hbm.at[row], osem)
        pltpu.async_copy(sibuf.at[p_], si_hbm.at[row], osem)

        nv = (row + 1) // _C
        @pl.when(nv != vprev)
        def _():
            pltpu.sync_copy(xt_hbm.at[jnp.minimum(nv, _V - 1)], xcol)

        return nv

    lax.fori_loop(0, _RPW, row_body, row0 // _C)

    last = row0 + _RPW - 1
    pltpu.make_async_copy(wbuf.at[(_RPW - 2) % 2], w_hbm.at[last - 1], osem).wait()
    pltpu.make_async_copy(sibuf.at[(_RPW - 2) % 2], si_hbm.at[last - 1], osem).wait()
    pltpu.make_async_copy(wbuf.at[(_RPW - 1) % 2], w_hbm.at[last], osem).wait()
    pltpu.make_async_copy(sibuf.at[(_RPW - 1) % 2], si_hbm.at[last], osem).wait()


_sc_lookup = functools.partial(
    pl.kernel,
    out_type=(
        jax.ShapeDtypeStruct((_ROWS, _B), jnp.float32),
        jax.ShapeDtypeStruct((_ROWS, _B), jnp.float32),
    ),
    mesh=plsc.VectorSubcoreMesh(core_axis_name="c", subcore_axis_name="s"),
    scratch_types=[
        pltpu.VMEM((_B,), jnp.int32),
        pltpu.VMEM((_H0,), jnp.float32),
        pltpu.VMEM((_H1,), jnp.float32),
        pltpu.VMEM((_B,), jnp.float32),
        pltpu.VMEM((2, _B), jnp.float32),
        pltpu.VMEM((2, _B), jnp.float32),
        pltpu.SemaphoreType.DMA,
        pltpu.SemaphoreType.DMA,
    ],
    compiler_params=pltpu.CompilerParams(needs_layout_passes=False),
)(_sc_body)


def kernel(x, weight):
    xt = x.T
    w2d = weight.reshape(_ROWS, _S)
    w, si = _sc_lookup(xt, w2d)
    w = w.T.reshape(_B, _V, 1, _C)
    si = si.T.reshape(_B, _V, 1, _C)
    return (w, si)
